# baseline (speedup 1.0000x reference)
import functools

import jax
import jax.numpy as jnp
from jax.experimental import pallas as pl
from jax.experimental.pallas import tpu as pltpu
from jax.experimental.pallas import tpu_sc as plsc

_NUM_E = 8192
_DIM = 256
_B = 2
_D = 8
_H = 32
_W = 32
_ROWS = _B * _D * _H * _W
_RB = 256
_HO = _H // 8
_NBLK = _ROWS // _RB
_COMMIT = 0.25


def _dist_kernel(x_ref, emb_ref, idx_ref, loss_ref, e2_ref, xcol_ref, xs_ref,
                 acc_ref):
    b, d, ho = pl.program_id(0), pl.program_id(1), pl.program_id(2)
    first = (b == 0) & (d == 0) & (ho == 0)

    @pl.when(first)
    def _init():
        emb = emb_ref[...]
        e2_ref[0, :] = jnp.sum(emb * emb, axis=1)
        acc_ref[...] = jnp.zeros((_RB, 1), jnp.float32)

    for h in range(8):
        xcol_ref[:, h * _W:(h + 1) * _W] = x_ref[0, :, 0, h, :]
    xs_ref[...] = jnp.transpose(xcol_ref[...])
    x = xs_ref[...]

    m = jax.lax.dot_general(
        x, emb_ref[...], (((1,), (1,)), ((), ())),
        preferred_element_type=jnp.float32,
        precision=jax.lax.Precision.DEFAULT)
    x2 = jnp.sum(x * x, axis=1, keepdims=True)
    dist = (x2 + e2_ref[0, :][None, :]) - 2.0 * m
    dmin = jnp.min(dist, axis=1, keepdims=True)
    ji = jax.lax.broadcasted_iota(jnp.int32, dist.shape, 1)
    idx = jnp.min(jnp.where(dist == dmin, ji, jnp.int32(_NUM_E)), axis=1)
    idx_ref[0, 0, :] = idx
    acc_ref[...] += dmin

    last = (b == _B - 1) & (d == _D - 1) & (ho == _HO - 1)

    @pl.when(last)
    def _fin():
        loss_ref[0, 0] = jnp.sum(acc_ref[...])


_dist_call = pl.pallas_call(
    _dist_kernel,
    grid=(_B, _D, _HO),
    in_specs=[
        pl.BlockSpec((1, _DIM, 1, 8, _W), lambda b, d, h: (b, 0, d, h, 0)),
        pl.BlockSpec((_NUM_E, _DIM), lambda b, d, h: (0, 0)),
    ],
    out_specs=[
        pl.BlockSpec((1, 1, _RB),
                     lambda b, d, h: (b * (_D * _HO) + d * _HO + h, 0, 0)),
        pl.BlockSpec(memory_space=pltpu.SMEM),
    ],
    out_shape=[
        jax.ShapeDtypeStruct((_NBLK, 1, _RB), jnp.int32),
        jax.ShapeDtypeStruct((1, 1), jnp.float32),
    ],
    scratch_shapes=[
        pltpu.VMEM((1, _NUM_E), jnp.float32),
        pltpu.VMEM((_DIM, _RB), jnp.float32),
        pltpu.VMEM((_RB, _DIM), jnp.float32),
        pltpu.VMEM((_RB, 1), jnp.float32),
    ],
)


def _repack_kernel(q_ref, o_ref, qt_ref):
    qt_ref[...] = jnp.transpose(q_ref[...])
    for h in range(8):
        o_ref[0, :, 0, h, :] = qt_ref[:, h * _W:(h + 1) * _W]


_repack_call = pl.pallas_call(
    _repack_kernel,
    grid=(_B, _D, _HO),
    in_specs=[
        pl.BlockSpec((_RB, _DIM),
                     lambda b, d, h: (b * (_D * _HO) + d * _HO + h, 0)),
    ],
    out_specs=pl.BlockSpec((1, _DIM, 1, 8, _W), lambda b, d, h: (b, 0, d, h, 0)),
    out_shape=jax.ShapeDtypeStruct((_B, _DIM, _D, _H, _W), jnp.float32),
    scratch_shapes=[pltpu.VMEM((_DIM, _RB), jnp.float32)],
)


def _sc_gather(embedding, idx3):
    @functools.partial(
        pl.kernel,
        out_type=jax.ShapeDtypeStruct((_ROWS, _DIM), jnp.float32),
        mesh=plsc.VectorSubcoreMesh(core_axis_name="c", subcore_axis_name="s"),
    )
    def gk(emb_hbm, i_hbm, o_hbm):
        def body(i_vmem, o_vmem):
            pltpu.sync_copy(emb_hbm.at[i_vmem.at[0, 0]], o_vmem)

        pltpu.emit_pipeline(
            body,
            grid=(_ROWS // 128,),
            in_specs=[pl.BlockSpec((1, 1, 128),
                                   index_map=lambda i: (i // 2, 0, i % 2))],
            out_specs=[pl.BlockSpec((128, _DIM), index_map=lambda i: (i, 0))],
            core_axis_name=("c", "s"),
            dimension_semantics=(pltpu.PARALLEL,),
        )(i_hbm, o_hbm)

    return gk(embedding, idx3)


def kernel(inputs, embedding):
    idx3, lsum = _dist_call(inputs, embedding)
    qflat = _sc_gather(embedding, idx3)
    quantized = _repack_call(qflat)
    a = lsum[0, 0] / jnp.float32(_ROWS * _DIM)
    loss = a + jnp.float32(_COMMIT) * a
    return quantized, loss
